# baseline (speedup 1.0000x reference)
import functools

import jax
import jax.numpy as jnp
from jax.experimental import pallas as pl
from jax.experimental.pallas import tpu as pltpu
from jax.experimental.pallas import tpu_sc as plsc

_N_FIELDS = 26
_VOCAB = 100000
_DIM = 16
_B = 16384
_D = _N_FIELDS * _DIM
_NUM_IDX = _B * _N_FIELDS

_GATHER_W = 128
_BS = 2048


def _format_body(t_ref, o_ref):
    chunk = 4096
    sub = chunk // 8
    rows = chunk * _DIM // 128
    n_c = 98304 // chunk
    rem = _VOCAB - n_c * chunk
    rem_rows = rem * _DIM // 128
    f_rows = _VOCAB * _DIM // 128

    for h in range(2):
        def step(c, carry, h=h):
            x = t_ref[h, :, pl.ds(c * chunk, chunk)]
            xx = jnp.concatenate(
                [x[:, j * sub:(j + 1) * sub] for j in range(8)], axis=0)
            o_ref[0, pl.ds(h * f_rows + c * rows, rows), :] = xx.T
            return carry

        jax.lax.fori_loop(0, n_c, step, 0)
        xr = t_ref[h, :, pl.ds(n_c * chunk, rem)]
        y3 = xr.T.reshape(rem // 8, 8, _DIM)
        packed = jnp.concatenate([y3[:, k, :] for k in range(8)], axis=1)
        o_ref[0, pl.ds(h * f_rows + n_c * rows, rem_rows), :] = packed


def _format_tables(tables_t, pair_off, n_pairs):
    return pl.pallas_call(
        _format_body,
        grid=(n_pairs,),
        in_specs=[pl.BlockSpec((2, _DIM, _VOCAB),
                               lambda f: (f + pair_off, 0, 0))],
        out_specs=pl.BlockSpec((1, 25000, 128), lambda f: (f, 0, 0)),
        out_shape=jax.ShapeDtypeStruct((n_pairs, 25000, 128), jnp.float32),
        compiler_params=pltpu.CompilerParams(
            dimension_semantics=("parallel",)),
    )(tables_t)


def _sc_gather(flat_tables, flat_idx, num_idx):
    mesh = plsc.VectorSubcoreMesh(core_axis_name="core", subcore_axis_name="subcore")
    idx2d = flat_idx.reshape(1, num_idx)

    @functools.partial(
        pl.kernel,
        out_type=jax.ShapeDtypeStruct((num_idx, _DIM), jnp.float32),
        mesh=mesh,
        compiler_params=pltpu.CompilerParams(use_tc_tiling_on_sc=False),
    )
    def gather_kernel(x_hbm, i_hbm, o_hbm):
        def body(i_vmem, o_vmem):
            pltpu.sync_copy(x_hbm.at[i_vmem.at[0]], o_vmem)

        pltpu.emit_pipeline(
            body,
            grid=(num_idx // _GATHER_W,),
            in_specs=[pl.BlockSpec((1, _GATHER_W), index_map=lambda i: (0, i))],
            out_specs=[pl.BlockSpec((_GATHER_W, _DIM), index_map=lambda i: (i, 0))],
            core_axis_name=("core", "subcore"),
            dimension_semantics=(pltpu.PARALLEL,),
        )(i_hbm, o_hbm)

    return gather_kernel(flat_tables, idx2d)


def _fold_bcast(row2, n):
    return jnp.broadcast_to(row2[None], (n, 2, 128)).reshape(2 * n, 128)


def _fold_sum(y):
    n2, m = y.shape
    y3 = y.reshape(n2 // 2, 2, m)
    return y3[:, 0, :] + y3[:, 1, :]


def _dense_body(xa_ref, xb_ref, cwf_ref, cb_ref, w1a_ref, w1b_ref, b1_ref,
                w2_ref, b2_ref, w3_ref, b3_ref, wff_ref, wfh_ref, bf_ref,
                out_ref):
    xa = xa_ref[...]
    xb = xb_ref[...]
    za, zb = xa, xb
    for l in range(3):
        wa = cwf_ref[l, 0:2, :]
        wb = cwf_ref[l, 2:4, :]
        t = za * _fold_bcast(wa, _BS)
        u = zb * _fold_bcast(wb, _BS)
        xw = _fold_sum(jnp.sum(t, axis=1, keepdims=True)
                       + jnp.sum(u, axis=1, keepdims=True))
        xwf = jnp.broadcast_to(xw[:, None, :], (_BS, 2, 1)).reshape(2 * _BS, 1)
        za = xa * xwf + _fold_bcast(cb_ref[l, 0:2, :], _BS) + za
        zb = xb * xwf + _fold_bcast(cb_ref[l, 2:4, :], _BS) + zb
    resa = jnp.dot(xa.astype(jnp.bfloat16), w1a_ref[...].astype(jnp.bfloat16),
                   preferred_element_type=jnp.float32)
    resb = jnp.dot(xb.astype(jnp.bfloat16), w1b_ref[...].astype(jnp.bfloat16),
                   preferred_element_type=jnp.float32)
    ya = resa.reshape(_BS, 2, 512)
    yb = resb.reshape(_BS, 2, 512)
    h = (ya[:, 0, 0:256] + ya[:, 1, 256:512]
         + yb[:, 0, 0:256] + yb[:, 1, 256:512] + b1_ref[0, :][None, :])
    h = jnp.maximum(h, 0.0)
    for w_ref, b_ref in ((w2_ref, b2_ref), (w3_ref, b3_ref)):
        h = jnp.maximum(
            jnp.dot(h.astype(jnp.bfloat16), w_ref[...].astype(jnp.bfloat16),
                    preferred_element_type=jnp.float32)
            + b_ref[0, :][None, :], 0.0)
    la = jnp.sum(za * _fold_bcast(wff_ref[0:2, :], _BS), axis=1, keepdims=True)
    lb = jnp.sum(zb * _fold_bcast(wff_ref[2:4, :], _BS), axis=1, keepdims=True)
    logit = (_fold_sum(la + lb)
             + jnp.dot(h, wfh_ref[...], preferred_element_type=jnp.float32)
             + bf_ref[0, 0])
    out_ref[...] = jax.nn.sigmoid(logit)


def _fold_feat(w, pad_a, pad_b):
    a = jnp.pad(w[..., :224], [(0, 0)] * (w.ndim - 1) + [(0, 32)])
    b = jnp.pad(w[..., 224:], [(0, 0)] * (w.ndim - 1) + [(0, 64)])
    sh = w.shape[:-1]
    return jnp.concatenate([a.reshape(sh + (2, 128)),
                            b.reshape(sh + (2, 128))], axis=-2)


def _fold_w1(Wpart, pad):
    Wp = jnp.pad(Wpart, ((0, pad), (0, 0)))
    return jnp.concatenate([Wp[:128], Wp[128:]], axis=1)


def _dense(xa, xb, cross_w, cross_b, W1, b1, W2, b2, W3, b3, Wf, bf):
    grid = (_B // _BS,)
    cwf = _fold_feat(cross_w, 32, 64)
    cbf = _fold_feat(cross_b, 32, 64)
    w1a = _fold_w1(W1[:224], 32)
    w1b = _fold_w1(W1[224:416], 64)
    wff = _fold_feat(Wf[:_D, 0], 32, 64)
    wfh = Wf[_D:, :]
    return pl.pallas_call(
        _dense_body,
        grid=grid,
        in_specs=[
            pl.BlockSpec((2 * _BS, 128), lambda i: (i, 0)),
            pl.BlockSpec((2 * _BS, 128), lambda i: (i, 0)),
            pl.BlockSpec((3, 4, 128), lambda i: (0, 0, 0)),
            pl.BlockSpec((3, 4, 128), lambda i: (0, 0, 0)),
            pl.BlockSpec((128, 512), lambda i: (0, 0)),
            pl.BlockSpec((128, 512), lambda i: (0, 0)),
            pl.BlockSpec((1, 256), lambda i: (0, 0)),
            pl.BlockSpec((256, 128), lambda i: (0, 0)),
            pl.BlockSpec((1, 128), lambda i: (0, 0)),
            pl.BlockSpec((128, 64), lambda i: (0, 0)),
            pl.BlockSpec((1, 64), lambda i: (0, 0)),
            pl.BlockSpec((4, 128), lambda i: (0, 0)),
            pl.BlockSpec((64, 1), lambda i: (0, 0)),
            pl.BlockSpec((1, 1), lambda i: (0, 0)),
        ],
        out_specs=pl.BlockSpec((_BS, 1), lambda i: (i, 0)),
        out_shape=jax.ShapeDtypeStruct((_B, 1), jnp.float32),
        compiler_params=pltpu.CompilerParams(
            dimension_semantics=("parallel",)),
    )(xa, xb, cwf, cbf, w1a, w1b, b1.reshape(1, 256), W2, b2.reshape(1, 128),
      W3, b3.reshape(1, 64), wff, wfh, bf.reshape(1, 1))


def kernel(inputs, tables, cross_w, cross_b, W1, b1, W2, b2, W3, b3, Wf, bf):
    v = inputs
    g = jnp.where(
        v < 98304,
        (v & ~4095) | ((v & 511) << 3) | ((v >> 9) & 7),
        v)
    offs = (jnp.arange(_N_FIELDS, dtype=jnp.int32) * _VOCAB)[None, :]
    n_fa = 14
    n_fb = _N_FIELDS - n_fa
    zero2 = jnp.zeros((_B, 16 - n_fa), dtype=jnp.int32)
    zero4 = jnp.zeros((_B, 16 - n_fb), dtype=jnp.int32)
    idx_a = jnp.concatenate([g[:, :n_fa] + offs[:, :n_fa], zero2], axis=1).reshape(-1)
    idx_b = jnp.concatenate([g[:, n_fa:] + offs[:, :n_fb], zero4], axis=1).reshape(-1)
    tables_t = jnp.transpose(tables, (0, 2, 1))
    flat_a = _format_tables(tables_t, 0, n_fa // 2).reshape(n_fa * _VOCAB, _DIM)
    flat_b = _format_tables(tables_t, n_fa // 2, n_fb // 2).reshape(n_fb * _VOCAB, _DIM)
    ga = _sc_gather(flat_a, idx_a, _B * 16)
    gb = _sc_gather(flat_b, idx_b, _B * 16)
    xa = ga.reshape(2 * _B, 128)
    xb = gb.reshape(2 * _B, 128)
    return _dense(xa, xb, cross_w, cross_b, W1, b1, W2, b2, W3, b3, Wf, bf)


# speedup vs baseline: 2.9150x; 2.9150x over previous
import functools

import jax
import jax.numpy as jnp
from jax.experimental import pallas as pl
from jax.experimental.pallas import tpu as pltpu
from jax.experimental.pallas import tpu_sc as plsc

_N_FIELDS = 26
_VOCAB = 100000
_DIM = 16
_B = 16384
_D = _N_FIELDS * _DIM
_NUM_IDX = _B * _N_FIELDS

_GATHER_W = 128
_BS = 2048


def _format_body(t_ref, o_ref):
    chunk = 4096
    sub = chunk // 8
    rows = chunk * _DIM // 128
    n_c = 98304 // chunk
    rem = _VOCAB - n_c * chunk
    rem_rows = rem * _DIM // 128
    f_rows = _VOCAB * _DIM // 128

    for h in range(2):
        def step(c, carry, h=h):
            x = t_ref[h, :, pl.ds(c * chunk, chunk)]
            xx = jnp.concatenate(
                [x[:, j * sub:(j + 1) * sub] for j in range(8)], axis=0)
            o_ref[0, pl.ds(h * f_rows + c * rows, rows), :] = xx.T
            return carry

        jax.lax.fori_loop(0, n_c, step, 0)
        xr = t_ref[h, :, pl.ds(n_c * chunk, rem)]
        y3 = xr.T.reshape(rem // 8, 8, _DIM)
        packed = jnp.concatenate([y3[:, k, :] for k in range(8)], axis=1)
        o_ref[0, pl.ds(h * f_rows + n_c * rows, rem_rows), :] = packed


def _format_tables(tables_t, pair_off, n_pairs):
    return pl.pallas_call(
        _format_body,
        grid=(n_pairs,),
        in_specs=[pl.BlockSpec((2, _DIM, _VOCAB),
                               lambda f: (f + pair_off, 0, 0))],
        out_specs=pl.BlockSpec((1, 25000, 128), lambda f: (f, 0, 0)),
        out_shape=jax.ShapeDtypeStruct((n_pairs, 25000, 128), jnp.float32),
        compiler_params=pltpu.CompilerParams(
            dimension_semantics=("parallel",)),
    )(tables_t)


def _sc_gather(flat_tables, flat_idx, num_idx):
    mesh = plsc.VectorSubcoreMesh(core_axis_name="core", subcore_axis_name="subcore")
    idx2d = flat_idx.reshape(1, num_idx)

    @functools.partial(
        pl.kernel,
        out_type=jax.ShapeDtypeStruct((num_idx, _DIM), jnp.float32),
        mesh=mesh,
        compiler_params=pltpu.CompilerParams(use_tc_tiling_on_sc=False),
    )
    def gather_kernel(x_hbm, i_hbm, o_hbm):
        def body(i_vmem, o_vmem):
            pltpu.sync_copy(x_hbm.at[i_vmem.at[0]], o_vmem)

        pltpu.emit_pipeline(
            body,
            grid=(num_idx // _GATHER_W,),
            in_specs=[pl.BlockSpec((1, _GATHER_W), index_map=lambda i: (0, i))],
            out_specs=[pl.BlockSpec((_GATHER_W, _DIM), index_map=lambda i: (i, 0))],
            core_axis_name=("core", "subcore"),
            dimension_semantics=(pltpu.PARALLEL,),
        )(i_hbm, o_hbm)

    return gather_kernel(flat_tables, idx2d)


def _dense_body(x0a_ref, x0b_ref, cw_ref, cb_ref, w1_ref, b1_ref, w2_ref, b2_ref,
                w3_ref, b3_ref, wf_ref, bf_ref, out_ref):
    x0 = jnp.concatenate([x0a_ref[...], x0b_ref[...]], axis=1)
    x = x0
    for l in range(3):
        w = cw_ref[l, :]
        xw = jnp.sum(x * w[None, :], axis=1, keepdims=True)
        x = x0 * xw + cb_ref[l, :][None, :] + x
    h = x0
    for w_ref, b_ref in ((w1_ref, b1_ref), (w2_ref, b2_ref), (w3_ref, b3_ref)):
        h = jnp.maximum(
            jnp.dot(h.astype(jnp.bfloat16), w_ref[...].astype(jnp.bfloat16),
                    preferred_element_type=jnp.float32)
            + b_ref[0, :][None, :], 0.0)
    wf = wf_ref[...]
    logit = (jnp.dot(x, wf[:_D, :], preferred_element_type=jnp.float32)
             + jnp.dot(h, wf[_D:, :], preferred_element_type=jnp.float32)
             + bf_ref[0, 0])
    out_ref[...] = jax.nn.sigmoid(logit)


def _dense(x0a, x0b, cross_w, cross_b, W1, b1, W2, b2, W3, b3, Wf, bf):
    grid = (_B // _BS,)
    return pl.pallas_call(
        _dense_body,
        grid=grid,
        in_specs=[
            pl.BlockSpec((_BS, x0a.shape[1]), lambda i: (i, 0)),
            pl.BlockSpec((_BS, x0b.shape[1]), lambda i: (i, 0)),
            pl.BlockSpec((3, _D), lambda i: (0, 0)),
            pl.BlockSpec((3, _D), lambda i: (0, 0)),
            pl.BlockSpec((_D, 256), lambda i: (0, 0)),
            pl.BlockSpec((1, 256), lambda i: (0, 0)),
            pl.BlockSpec((256, 128), lambda i: (0, 0)),
            pl.BlockSpec((1, 128), lambda i: (0, 0)),
            pl.BlockSpec((128, 64), lambda i: (0, 0)),
            pl.BlockSpec((1, 64), lambda i: (0, 0)),
            pl.BlockSpec((_D + 64, 1), lambda i: (0, 0)),
            pl.BlockSpec((1, 1), lambda i: (0, 0)),
        ],
        out_specs=pl.BlockSpec((_BS, 1), lambda i: (i, 0)),
        out_shape=jax.ShapeDtypeStruct((_B, 1), jnp.float32),
        compiler_params=pltpu.CompilerParams(
            dimension_semantics=("parallel",)),
    )(x0a, x0b, cross_w, cross_b, W1, b1.reshape(1, 256), W2, b2.reshape(1, 128),
      W3, b3.reshape(1, 64), Wf, bf.reshape(1, 1))


def kernel(inputs, tables, cross_w, cross_b, W1, b1, W2, b2, W3, b3, Wf, bf):
    v = inputs
    g = jnp.where(
        v < 98304,
        (v & ~4095) | ((v & 511) << 3) | ((v >> 9) & 7),
        v)
    offs = (jnp.arange(_N_FIELDS, dtype=jnp.int32) * _VOCAB)[None, :]
    n_fa = 14
    n_fb = _N_FIELDS - n_fa
    idx_a = (g[:, :n_fa] + offs[:, :n_fa]).reshape(-1)
    idx_b = (g[:, n_fa:] + offs[:, :n_fb]).reshape(-1)
    tables_t = jnp.transpose(tables, (0, 2, 1))
    flat_a = _format_tables(tables_t, 0, n_fa // 2).reshape(n_fa * _VOCAB, _DIM)
    flat_b = _format_tables(tables_t, n_fa // 2, n_fb // 2).reshape(n_fb * _VOCAB, _DIM)
    ga = _sc_gather(flat_a, idx_a, _B * n_fa)
    gb = _sc_gather(flat_b, idx_b, _B * n_fb)
    x0a = ga.reshape(_B, n_fa * _DIM)
    x0b = gb.reshape(_B, n_fb * _DIM)
    return _dense(x0a, x0b, cross_w, cross_b, W1, b1, W2, b2, W3, b3, Wf, bf)
